# initial kernel scaffold (unmeasured)
import jax
import jax.numpy as jnp
from jax import lax
from jax.experimental import pallas as pl
from jax.experimental.pallas import tpu as pltpu

T = 4096
N = 1024
R = 512
K_MAX = T // R


def kernel(x, dest):
    my_x = lax.axis_index("x")

    c0 = jnp.sum(dest == 0).astype(jnp.int32)
    order = jnp.argsort(dest, stable=True)
    xs = jnp.take(x, order, axis=0)

    is0 = my_x == 0
    L = jnp.where(is0, T - c0, c0).astype(jnp.int32)
    keep_off = jnp.where(is0, 0, c0).astype(jnp.int32)
    keep_len = (T - L).astype(jnp.int32)
    send_src = jnp.where(is0, c0, 0).astype(jnp.int32)
    remote_dst = jnp.where(is0, 0, T - c0).astype(jnp.int32)
    recv_off = jnp.where(is0, c0, 0).astype(jnp.int32)
    k_send = (L + R - 1) // R
    k_keep = (keep_len + R - 1) // R

    meta = jnp.stack(
        [keep_off, keep_len, send_src, remote_dst, recv_off, L, k_send, k_keep]
    ).astype(jnp.int32)

    def body(meta_ref, xs_ref, out_ref, send_sems, recv_sems, copy_sems):
        nbr = (1 - lax.axis_index("x"), lax.axis_index("y"))

        barrier = pltpu.get_barrier_semaphore()
        pl.semaphore_signal(
            barrier, inc=1, device_id=nbr, device_id_type=pl.DeviceIdType.MESH
        )
        pl.semaphore_wait(barrier, 1)

        keep_off_ = meta_ref[0]
        keep_len_ = meta_ref[1]
        send_src_ = meta_ref[2]
        remote_dst_ = meta_ref[3]
        recv_off_ = meta_ref[4]
        L_ = meta_ref[5]
        k_send_ = meta_ref[6]
        k_keep_ = meta_ref[7]

        def start_of(i, length):
            return jnp.minimum(i * R, length - R)

        def send_rdma(i):
            s = start_of(i, L_)
            return pltpu.make_async_remote_copy(
                src_ref=xs_ref.at[pl.ds(send_src_ + s, R)],
                dst_ref=out_ref.at[pl.ds(remote_dst_ + s, R)],
                send_sem=send_sems.at[i],
                recv_sem=recv_sems.at[i],
                device_id=nbr,
                device_id_type=pl.DeviceIdType.MESH,
            )

        def recv_rdma(i):
            s = start_of(i, L_)
            return pltpu.make_async_remote_copy(
                src_ref=xs_ref.at[pl.ds(send_src_ + s, R)],
                dst_ref=out_ref.at[pl.ds(recv_off_ + s, R)],
                send_sem=send_sems.at[i],
                recv_sem=recv_sems.at[i],
                device_id=nbr,
                device_id_type=pl.DeviceIdType.MESH,
            )

        def keep_dma(i):
            s = keep_off_ + start_of(i, keep_len_)
            return pltpu.make_async_copy(
                xs_ref.at[pl.ds(s, R)],
                out_ref.at[pl.ds(s, R)],
                copy_sems.at[i],
            )

        for i in range(K_MAX):
            @pl.when(i < k_send_)
            def _():
                send_rdma(i).start()

        for i in range(K_MAX):
            @pl.when(i < k_keep_)
            def _():
                keep_dma(i).start()

        for i in range(K_MAX):
            @pl.when(i < k_keep_)
            def _():
                keep_dma(i).wait()

        for i in range(K_MAX):
            @pl.when(i < k_send_)
            def _():
                recv_rdma(i).wait_recv()

        for i in range(K_MAX):
            @pl.when(i < k_send_)
            def _():
                send_rdma(i).wait_send()

    return pl.pallas_call(
        body,
        out_shape=jax.ShapeDtypeStruct((T, N), jnp.float32),
        in_specs=[
            pl.BlockSpec(memory_space=pltpu.SMEM),
            pl.BlockSpec(memory_space=pltpu.VMEM),
        ],
        out_specs=pl.BlockSpec(memory_space=pltpu.VMEM),
        scratch_shapes=[
            pltpu.SemaphoreType.DMA((K_MAX,)),
            pltpu.SemaphoreType.DMA((K_MAX,)),
            pltpu.SemaphoreType.DMA((K_MAX,)),
        ],
        compiler_params=pltpu.CompilerParams(collective_id=0),
    )(meta, xs)


# baseline (device time: 277894 ns/iter reference)
import jax
import jax.numpy as jnp
from jax import lax
from jax.experimental import pallas as pl
from jax.experimental.pallas import tpu as pltpu

T = 4096
N = 1024
R = 512
K_MAX = T // R


def kernel(x, dest):
    my_x = lax.axis_index("x")

    c0 = jnp.sum(dest == 0).astype(jnp.int32)
    order = jnp.argsort(dest, stable=True)
    xs = jnp.take(x, order, axis=0)
    xs = xs.reshape(T, 8, N // 8)

    is0 = my_x == 0
    L = jnp.where(is0, T - c0, c0).astype(jnp.int32)
    keep_off = jnp.where(is0, 0, c0).astype(jnp.int32)
    keep_len = (T - L).astype(jnp.int32)
    send_src = jnp.where(is0, c0, 0).astype(jnp.int32)
    remote_dst = jnp.where(is0, 0, T - c0).astype(jnp.int32)
    recv_off = jnp.where(is0, c0, 0).astype(jnp.int32)
    k_send = (L + R - 1) // R
    k_keep = (keep_len + R - 1) // R

    meta = jnp.stack(
        [keep_off, keep_len, send_src, remote_dst, recv_off, L, k_send, k_keep]
    ).astype(jnp.int32)

    def body(meta_ref, xs_ref, out_ref, send_sems, recv_sems, copy_sems):
        nbr = (1 - lax.axis_index("x"), lax.axis_index("y"))

        barrier = pltpu.get_barrier_semaphore()
        pl.semaphore_signal(
            barrier, inc=1, device_id=nbr, device_id_type=pl.DeviceIdType.MESH
        )
        pl.semaphore_wait(barrier, 1)

        keep_off_ = meta_ref[0]
        keep_len_ = meta_ref[1]
        send_src_ = meta_ref[2]
        remote_dst_ = meta_ref[3]
        recv_off_ = meta_ref[4]
        L_ = meta_ref[5]
        k_send_ = meta_ref[6]
        k_keep_ = meta_ref[7]

        def start_of(i, length):
            return jnp.minimum(i * R, length - R)

        def send_rdma(i):
            s = start_of(i, L_)
            return pltpu.make_async_remote_copy(
                src_ref=xs_ref.at[pl.ds(send_src_ + s, R)],
                dst_ref=out_ref.at[pl.ds(remote_dst_ + s, R)],
                send_sem=send_sems.at[i],
                recv_sem=recv_sems.at[i],
                device_id=nbr,
                device_id_type=pl.DeviceIdType.MESH,
            )

        def recv_rdma(i):
            s = start_of(i, L_)
            return pltpu.make_async_remote_copy(
                src_ref=xs_ref.at[pl.ds(send_src_ + s, R)],
                dst_ref=out_ref.at[pl.ds(recv_off_ + s, R)],
                send_sem=send_sems.at[i],
                recv_sem=recv_sems.at[i],
                device_id=nbr,
                device_id_type=pl.DeviceIdType.MESH,
            )

        def keep_dma(i):
            s = keep_off_ + start_of(i, keep_len_)
            return pltpu.make_async_copy(
                xs_ref.at[pl.ds(s, R)],
                out_ref.at[pl.ds(s, R)],
                copy_sems.at[i],
            )

        for i in range(K_MAX):
            @pl.when(i < k_send_)
            def _():
                send_rdma(i).start()

        for i in range(K_MAX):
            @pl.when(i < k_keep_)
            def _():
                keep_dma(i).start()

        for i in range(K_MAX):
            @pl.when(i < k_keep_)
            def _():
                keep_dma(i).wait()

        for i in range(K_MAX):
            @pl.when(i < k_send_)
            def _():
                recv_rdma(i).wait_recv()

        for i in range(K_MAX):
            @pl.when(i < k_send_)
            def _():
                send_rdma(i).wait_send()

    out = pl.pallas_call(
        body,
        out_shape=jax.ShapeDtypeStruct((T, 8, N // 8), jnp.float32),
        in_specs=[
            pl.BlockSpec(memory_space=pltpu.SMEM),
            pl.BlockSpec(memory_space=pltpu.VMEM),
        ],
        out_specs=pl.BlockSpec(memory_space=pltpu.VMEM),
        scratch_shapes=[
            pltpu.SemaphoreType.DMA((K_MAX,)),
            pltpu.SemaphoreType.DMA((K_MAX,)),
            pltpu.SemaphoreType.DMA((K_MAX,)),
        ],
        compiler_params=pltpu.CompilerParams(collective_id=0),
    )(meta, xs)
    return out.reshape(T, N)


# device time: 155357 ns/iter; 1.7887x vs baseline; 1.7887x over previous
import jax
import jax.numpy as jnp
from jax import lax
from jax.experimental import pallas as pl
from jax.experimental.pallas import tpu as pltpu

T = 4096
N = 1024


def kernel(x, dest):
    my_x = lax.axis_index("x")
    dest = dest.astype(jnp.int32)

    ones_before = (jnp.cumsum(dest) - dest).astype(jnp.int32)
    zeros_before = (jnp.arange(T, dtype=jnp.int32) - ones_before)
    before = jnp.where(dest == 0, zeros_before, ones_before)

    s1 = ones_before[-1] + dest[-1]
    c0 = T - s1
    is0 = my_x == 0
    L = jnp.where(is0, s1, c0).astype(jnp.int32)
    base_keep = jnp.where(is0, 0, c0).astype(jnp.int32)
    base_send = jnp.where(is0, 0, T - c0).astype(jnp.int32)

    pos = (jnp.where(dest == my_x, base_keep, base_send) + before).astype(
        jnp.int32
    )

    meta = jnp.stack([L, T - L, my_x.astype(jnp.int32)])
    x3 = x.reshape(T, 8, N // 8)

    def body(meta_ref, dest_ref, pos_ref, x_ref, out_ref,
             send_sem, recv_sem, copy_sem):
        nbr = (1 - lax.axis_index("x"), lax.axis_index("y"))

        barrier = pltpu.get_barrier_semaphore()
        pl.semaphore_signal(
            barrier, inc=1, device_id=nbr, device_id_type=pl.DeviceIdType.MESH
        )
        pl.semaphore_wait(barrier, 1)

        L_ = meta_ref[0]
        keep_ = meta_ref[1]
        rank_ = meta_ref[2]

        def row(j, _):
            d = dest_ref[j]
            p = pos_ref[j]

            @pl.when(d == rank_)
            def _():
                pltpu.make_async_copy(
                    x_ref.at[pl.ds(j, 1)],
                    out_ref.at[pl.ds(p, 1)],
                    copy_sem,
                ).start()

            @pl.when(d != rank_)
            def _():
                pltpu.make_async_remote_copy(
                    src_ref=x_ref.at[pl.ds(j, 1)],
                    dst_ref=out_ref.at[pl.ds(p, 1)],
                    send_sem=send_sem,
                    recv_sem=recv_sem,
                    device_id=nbr,
                    device_id_type=pl.DeviceIdType.MESH,
                ).start()

            return 0

        lax.fori_loop(0, T, row, 0)

        def wait_keep(_, __):
            pltpu.make_async_copy(
                x_ref.at[pl.ds(0, 1)], out_ref.at[pl.ds(0, 1)], copy_sem
            ).wait()
            return 0

        def wait_remote(kind):
            def w(_, __):
                d = pltpu.make_async_remote_copy(
                    src_ref=x_ref.at[pl.ds(0, 1)],
                    dst_ref=out_ref.at[pl.ds(0, 1)],
                    send_sem=send_sem,
                    recv_sem=recv_sem,
                    device_id=nbr,
                    device_id_type=pl.DeviceIdType.MESH,
                )
                if kind == "recv":
                    d.wait_recv()
                else:
                    d.wait_send()
                return 0
            return w

        lax.fori_loop(0, keep_, wait_keep, 0)
        lax.fori_loop(0, L_, wait_remote("recv"), 0)
        lax.fori_loop(0, L_, wait_remote("send"), 0)

    out = pl.pallas_call(
        body,
        out_shape=jax.ShapeDtypeStruct((T, 8, N // 8), jnp.float32),
        in_specs=[
            pl.BlockSpec(memory_space=pltpu.SMEM),
            pl.BlockSpec(memory_space=pltpu.SMEM),
            pl.BlockSpec(memory_space=pltpu.SMEM),
            pl.BlockSpec(memory_space=pltpu.VMEM),
        ],
        out_specs=pl.BlockSpec(memory_space=pltpu.VMEM),
        scratch_shapes=[
            pltpu.SemaphoreType.DMA,
            pltpu.SemaphoreType.DMA,
            pltpu.SemaphoreType.DMA,
        ],
        compiler_params=pltpu.CompilerParams(collective_id=0),
    )(meta, dest, pos, x3)
    return out.reshape(T, N)


# device time: 141930 ns/iter; 1.9580x vs baseline; 1.0946x over previous
import jax
import jax.numpy as jnp
from jax import lax
from jax.experimental import pallas as pl
from jax.experimental.pallas import tpu as pltpu

T = 4096
N = 1024


def kernel(x, dest):
    my_x = lax.axis_index("x")
    dest = dest.astype(jnp.int32)

    order = jnp.argsort(dest, stable=True).astype(jnp.int32)
    c0 = jnp.sum(dest == 0).astype(jnp.int32)

    is0 = my_x == 0
    L = jnp.where(is0, T - c0, c0).astype(jnp.int32)
    base_keep = jnp.where(is0, 0, c0).astype(jnp.int32)
    base_send = jnp.where(is0, 0, T - c0).astype(jnp.int32)
    ko = jnp.where(is0, 0, c0).astype(jnp.int32)
    so = jnp.where(is0, c0, 0).astype(jnp.int32)

    meta = jnp.stack([L, T - L, base_keep, base_send, ko, so])
    x3 = x.reshape(T, 8, N // 8)

    def body(meta_ref, order_ref, x_ref, out_ref, send_sem, recv_sem, copy_sem):
        nbr = (1 - lax.axis_index("x"), lax.axis_index("y"))

        barrier = pltpu.get_barrier_semaphore()
        pl.semaphore_signal(
            barrier, inc=1, device_id=nbr, device_id_type=pl.DeviceIdType.MESH
        )
        pl.semaphore_wait(barrier, 1)

        L_ = meta_ref[0]
        keep_ = meta_ref[1]
        base_keep_ = meta_ref[2]
        base_send_ = meta_ref[3]
        ko_ = meta_ref[4]
        so_ = meta_ref[5]

        def send_row(i, _):
            src = order_ref[so_ + i]
            pltpu.make_async_remote_copy(
                src_ref=x_ref.at[pl.ds(src, 1)],
                dst_ref=out_ref.at[pl.ds(base_send_ + i, 1)],
                send_sem=send_sem,
                recv_sem=recv_sem,
                device_id=nbr,
                device_id_type=pl.DeviceIdType.MESH,
            ).start()
            return 0

        def keep_row(i, _):
            src = order_ref[ko_ + i]
            pltpu.make_async_copy(
                x_ref.at[pl.ds(src, 1)],
                out_ref.at[pl.ds(base_keep_ + i, 1)],
                copy_sem,
            ).start()
            return 0

        lax.fori_loop(0, L_, send_row, 0)
        lax.fori_loop(0, keep_, keep_row, 0)

        def wait_keep(_, __):
            pltpu.make_async_copy(
                x_ref.at[pl.ds(0, 1)], out_ref.at[pl.ds(0, 1)], copy_sem
            ).wait()
            return 0

        def wait_remote(kind):
            def w(_, __):
                d = pltpu.make_async_remote_copy(
                    src_ref=x_ref.at[pl.ds(0, 1)],
                    dst_ref=out_ref.at[pl.ds(0, 1)],
                    send_sem=send_sem,
                    recv_sem=recv_sem,
                    device_id=nbr,
                    device_id_type=pl.DeviceIdType.MESH,
                )
                if kind == "recv":
                    d.wait_recv()
                else:
                    d.wait_send()
                return 0
            return w

        lax.fori_loop(0, keep_, wait_keep, 0)
        lax.fori_loop(0, L_, wait_remote("recv"), 0)
        lax.fori_loop(0, L_, wait_remote("send"), 0)

    out = pl.pallas_call(
        body,
        out_shape=jax.ShapeDtypeStruct((T, 8, N // 8), jnp.float32),
        in_specs=[
            pl.BlockSpec(memory_space=pltpu.SMEM),
            pl.BlockSpec(memory_space=pltpu.SMEM),
            pl.BlockSpec(memory_space=pltpu.VMEM),
        ],
        out_specs=pl.BlockSpec(memory_space=pltpu.VMEM),
        scratch_shapes=[
            pltpu.SemaphoreType.DMA,
            pltpu.SemaphoreType.DMA,
            pltpu.SemaphoreType.DMA,
        ],
        compiler_params=pltpu.CompilerParams(collective_id=0),
    )(meta, order, x3)
    return out.reshape(T, N)


# device time: 130590 ns/iter; 2.1280x vs baseline; 1.0868x over previous
import jax
import jax.numpy as jnp
from jax import lax
from jax.experimental import pallas as pl
from jax.experimental.pallas import tpu as pltpu

T = 4096
N = 1024
U = 4


def kernel(x, dest):
    my_x = lax.axis_index("x")
    dest = dest.astype(jnp.int32)

    order = jnp.argsort(dest, stable=True).astype(jnp.int32)
    c0 = jnp.sum(dest == 0).astype(jnp.int32)

    is0 = my_x == 0
    L = jnp.where(is0, T - c0, c0).astype(jnp.int32)
    keep = (T - L).astype(jnp.int32)
    base_keep = jnp.where(is0, 0, c0).astype(jnp.int32)
    base_send = jnp.where(is0, 0, T - c0).astype(jnp.int32)
    ko = jnp.where(is0, 0, c0).astype(jnp.int32)
    so = jnp.where(is0, c0, 0).astype(jnp.int32)

    def chunks(n):
        return [n // 512, (n % 512) // 8, n % 8]

    meta = jnp.stack(
        [L, keep, base_keep, base_send, ko, so, *chunks(L), *chunks(keep)]
    )
    x3 = x.reshape(T, 8, N // 8)

    def body(meta_ref, order_ref, x_ref, out_ref, send_sem, recv_sem, copy_sem):
        nbr = (1 - lax.axis_index("x"), lax.axis_index("y"))

        barrier = pltpu.get_barrier_semaphore()
        pl.semaphore_signal(
            barrier, inc=1, device_id=nbr, device_id_type=pl.DeviceIdType.MESH
        )
        pl.semaphore_wait(barrier, 1)

        L_ = meta_ref[0]
        keep_ = meta_ref[1]
        base_keep_ = meta_ref[2]
        base_send_ = meta_ref[3]
        ko_ = meta_ref[4]
        so_ = meta_ref[5]

        def send_one(i):
            src = order_ref[so_ + i]
            pltpu.make_async_remote_copy(
                src_ref=x_ref.at[pl.ds(src, 1)],
                dst_ref=out_ref.at[pl.ds(base_send_ + i, 1)],
                send_sem=send_sem,
                recv_sem=recv_sem,
                device_id=nbr,
                device_id_type=pl.DeviceIdType.MESH,
            ).start()

        def keep_one(i):
            src = order_ref[ko_ + i]
            pltpu.make_async_copy(
                x_ref.at[pl.ds(src, 1)],
                out_ref.at[pl.ds(base_keep_ + i, 1)],
                copy_sem,
            ).start()

        def unrolled(issue, count):
            def block(t, _):
                for u in range(U):
                    issue(t * U + u)
                return 0

            def tail(i, _):
                issue(i)
                return 0

            lax.fori_loop(0, count // U, block, 0)
            lax.fori_loop((count // U) * U, count, tail, 0)

        unrolled(send_one, L_)
        unrolled(keep_one, keep_)

        def drain(sem_wait, counts_at):
            for sz, k in zip((512, 8, 1), counts_at):
                def w(_, __, sz=sz):
                    sem_wait(pl.ds(0, sz))
                    return 0
                lax.fori_loop(0, meta_ref[k], w, 0)

        def wait_copy(sl):
            pltpu.make_async_copy(
                x_ref.at[sl], out_ref.at[sl], copy_sem
            ).wait()

        def wait_remote(kind):
            def w(sl):
                d = pltpu.make_async_remote_copy(
                    src_ref=x_ref.at[sl],
                    dst_ref=out_ref.at[sl],
                    send_sem=send_sem,
                    recv_sem=recv_sem,
                    device_id=nbr,
                    device_id_type=pl.DeviceIdType.MESH,
                )
                if kind == "recv":
                    d.wait_recv()
                else:
                    d.wait_send()
            return w

        drain(wait_copy, (9, 10, 11))
        drain(wait_remote("recv"), (6, 7, 8))
        drain(wait_remote("send"), (6, 7, 8))

    out = pl.pallas_call(
        body,
        out_shape=jax.ShapeDtypeStruct((T, 8, N // 8), jnp.float32),
        in_specs=[
            pl.BlockSpec(memory_space=pltpu.SMEM),
            pl.BlockSpec(memory_space=pltpu.SMEM),
            pl.BlockSpec(memory_space=pltpu.VMEM),
        ],
        out_specs=pl.BlockSpec(memory_space=pltpu.VMEM),
        scratch_shapes=[
            pltpu.SemaphoreType.DMA,
            pltpu.SemaphoreType.DMA,
            pltpu.SemaphoreType.DMA,
        ],
        compiler_params=pltpu.CompilerParams(collective_id=0),
    )(meta, order, x3)
    return out.reshape(T, N)


# device time: 108087 ns/iter; 2.5710x vs baseline; 1.2082x over previous
import jax
import jax.numpy as jnp
from jax import lax
from jax.experimental import pallas as pl
from jax.experimental.pallas import tpu as pltpu

T = 4096
N = 1024
U = 4


def kernel(x, dest):
    my_x = lax.axis_index("x")
    dest = dest.astype(jnp.int32)

    order = jnp.argsort(dest, stable=True).astype(jnp.int32)
    c0 = jnp.sum(dest == 0).astype(jnp.int32)

    is0 = my_x == 0
    L = jnp.where(is0, T - c0, c0).astype(jnp.int32)
    keep = (T - L).astype(jnp.int32)
    base_keep = jnp.where(is0, 0, c0).astype(jnp.int32)
    base_send = jnp.where(is0, 0, T - c0).astype(jnp.int32)
    ko = jnp.where(is0, 0, c0).astype(jnp.int32)
    so = jnp.where(is0, c0, 0).astype(jnp.int32)

    def chunks(n):
        return [n // 512, (n % 512) // 8, n % 8]

    meta = jnp.stack(
        [L, keep, base_keep, base_send, ko, so, *chunks(L), *chunks(keep)]
    )
    x4 = x.reshape(T // 8, 8, 8, N // 8).transpose(0, 2, 1, 3)

    def body(meta_ref, order_ref, x_ref, out_ref, send_sem, recv_sem, copy_sem):
        nbr = (1 - lax.axis_index("x"), lax.axis_index("y"))

        barrier = pltpu.get_barrier_semaphore()
        pl.semaphore_signal(
            barrier, inc=1, device_id=nbr, device_id_type=pl.DeviceIdType.MESH
        )
        pl.semaphore_wait(barrier, 1)

        L_ = meta_ref[0]
        keep_ = meta_ref[1]
        base_keep_ = meta_ref[2]
        base_send_ = meta_ref[3]
        ko_ = meta_ref[4]
        so_ = meta_ref[5]

        def row_slice(ref, j):
            return ref.at[pl.ds(j // 8, 1), :, pl.ds(j % 8, 1), :]

        def send_one(i):
            src = order_ref[so_ + i]
            pltpu.make_async_remote_copy(
                src_ref=row_slice(x_ref, src),
                dst_ref=row_slice(out_ref, base_send_ + i),
                send_sem=send_sem,
                recv_sem=recv_sem,
                device_id=nbr,
                device_id_type=pl.DeviceIdType.MESH,
            ).start()

        def keep_one(i):
            src = order_ref[ko_ + i]
            pltpu.make_async_copy(
                row_slice(x_ref, src),
                row_slice(out_ref, base_keep_ + i),
                copy_sem,
            ).start()

        def unrolled(issue, count):
            def block(t, _):
                for u in range(U):
                    issue(t * U + u)
                return 0

            def tail(i, _):
                issue(i)
                return 0

            lax.fori_loop(0, count // U, block, 0)
            lax.fori_loop((count // U) * U, count, tail, 0)

        unrolled(send_one, L_)
        unrolled(keep_one, keep_)

        def drain(sem_wait, counts_at):
            for sz, k in zip((512, 8, 1), counts_at):
                def w(_, __, sz=sz):
                    sem_wait((pl.ds(0, sz), slice(None), pl.ds(0, 1)))
                    return 0
                lax.fori_loop(0, meta_ref[k], w, 0)

        def wait_copy(sl):
            pltpu.make_async_copy(
                x_ref.at[sl], out_ref.at[sl], copy_sem
            ).wait()

        def wait_remote(kind):
            def w(sl):
                d = pltpu.make_async_remote_copy(
                    src_ref=x_ref.at[sl],
                    dst_ref=out_ref.at[sl],
                    send_sem=send_sem,
                    recv_sem=recv_sem,
                    device_id=nbr,
                    device_id_type=pl.DeviceIdType.MESH,
                )
                if kind == "recv":
                    d.wait_recv()
                else:
                    d.wait_send()
            return w

        drain(wait_copy, (9, 10, 11))
        drain(wait_remote("recv"), (6, 7, 8))
        drain(wait_remote("send"), (6, 7, 8))

    out = pl.pallas_call(
        body,
        out_shape=jax.ShapeDtypeStruct((T // 8, 8, 8, N // 8), jnp.float32),
        in_specs=[
            pl.BlockSpec(memory_space=pltpu.SMEM),
            pl.BlockSpec(memory_space=pltpu.SMEM),
            pl.BlockSpec(memory_space=pltpu.HBM),
        ],
        out_specs=pl.BlockSpec(memory_space=pltpu.HBM),
        scratch_shapes=[
            pltpu.SemaphoreType.DMA,
            pltpu.SemaphoreType.DMA,
            pltpu.SemaphoreType.DMA,
        ],
        compiler_params=pltpu.CompilerParams(collective_id=0),
    )(meta, order, x4)
    return out.transpose(0, 2, 1, 3).reshape(T, N)
